# Initial kernel scaffold; baseline (speedup 1.0000x reference)
import jax
import jax.numpy as jnp
from jax.experimental import pallas as pl


def kernel(input, node_segment):
    raise NotImplementedError("write your pallas kernel here")


# speedup vs baseline: 1.2540x; 1.2540x over previous
import functools

import jax
import jax.numpy as jnp
from jax import lax
from jax.experimental import pallas as pl
from jax.experimental.pallas import tpu as pltpu
from jax.experimental.pallas import tpu_sc as plsc

NUM_CORES = 2
NUM_SUBCORES = 16
NUM_WORKERS = NUM_CORES * NUM_SUBCORES
CHUNK = 400


def kernel(input, node_segment):
    n = node_segment.shape[0]
    d = input.shape[1]
    num_chunks = n // CHUNK
    assert num_chunks * CHUNK == n
    max_k = -(-num_chunks // NUM_WORKERS)

    idx = node_segment.astype(jnp.int32)
    mesh = plsc.VectorSubcoreMesh(core_axis_name="c", subcore_axis_name="s")

    @functools.partial(
        pl.kernel,
        out_type=jax.ShapeDtypeStruct((n, d), input.dtype),
        mesh=mesh,
        scratch_types=[
            pltpu.VMEM((CHUNK,), jnp.int32),
            pltpu.VMEM((CHUNK, d), jnp.float32),
            pltpu.SemaphoreType.DMA,
        ],
    )
    def gather_kernel(table_hbm, idx_hbm, out_hbm, idx_v, rows_v, sem):
        wid = lax.axis_index("s") * NUM_CORES + lax.axis_index("c")

        @pl.loop(0, max_k)
        def _(k):
            cid = wid + k * NUM_WORKERS

            @pl.when(cid < num_chunks)
            def _():
                base = cid * CHUNK
                pltpu.sync_copy(idx_hbm.at[pl.ds(base, CHUNK)], idx_v)
                pltpu.async_copy(table_hbm.at[idx_v], rows_v, sem).wait()
                pltpu.sync_copy(rows_v, out_hbm.at[pl.ds(base, CHUNK)])

    return gather_kernel(input, idx)


# speedup vs baseline: 1.3034x; 1.0394x over previous
import functools

import jax
import jax.numpy as jnp
from jax import lax
from jax.experimental import pallas as pl
from jax.experimental.pallas import tpu as pltpu
from jax.experimental.pallas import tpu_sc as plsc

NUM_CORES = 2
NUM_SUBCORES = 16
NUM_WORKERS = NUM_CORES * NUM_SUBCORES
CHUNK = 400


def kernel(input, node_segment):
    n = node_segment.shape[0]
    d = input.shape[1]
    num_chunks = n // CHUNK
    assert num_chunks * CHUNK == n
    max_k = -(-num_chunks // NUM_WORKERS)
    assert num_chunks // NUM_WORKERS >= 2

    idx = node_segment.astype(jnp.int32)
    mesh = plsc.VectorSubcoreMesh(core_axis_name="c", subcore_axis_name="s")

    @functools.partial(
        pl.kernel,
        out_type=jax.ShapeDtypeStruct((n, d), input.dtype),
        mesh=mesh,
        scratch_types=[
            pltpu.VMEM((CHUNK,), jnp.int32),
            pltpu.VMEM((CHUNK,), jnp.int32),
            pltpu.VMEM((CHUNK, d), jnp.float32),
            pltpu.VMEM((CHUNK, d), jnp.float32),
            pltpu.SemaphoreType.DMA,
            pltpu.SemaphoreType.DMA,
            pltpu.SemaphoreType.DMA,
            pltpu.SemaphoreType.DMA,
        ],
    )
    def gather_kernel(table_hbm, idx_hbm, out_hbm, idx_v0, idx_v1,
                      rows_v0, rows_v1, sem_g0, sem_g1, sem_w0, sem_w1):
        wid = lax.axis_index("s") * NUM_CORES + lax.axis_index("c")
        idx_v = (idx_v0, idx_v1)
        rows_v = (rows_v0, rows_v1)
        sem_g = (sem_g0, sem_g1)
        sem_w = (sem_w0, sem_w1)

        def chunk_base(k):
            return (wid + k * NUM_WORKERS) * CHUNK

        def have_chunk(k):
            return (k + 1) * NUM_WORKERS <= num_chunks

        def guarded(k, fn):
            if have_chunk(k):
                fn()
            else:
                pl.when(wid + k * NUM_WORKERS < num_chunks)(fn)

        def load_and_gather(k):
            b = k & 1
            pltpu.sync_copy(idx_hbm.at[pl.ds(chunk_base(k), CHUNK)], idx_v[b])
            if k >= 2:
                pltpu.make_async_copy(
                    rows_v[b], out_hbm.at[pl.ds(0, CHUNK)], sem_w[b]).wait()
            pltpu.async_copy(table_hbm.at[idx_v[b]], rows_v[b], sem_g[b])

        def finish_chunk(k):
            b = k & 1
            pltpu.make_async_copy(
                table_hbm.at[idx_v[b]], rows_v[b], sem_g[b]).wait()
            pltpu.async_copy(
                rows_v[b], out_hbm.at[pl.ds(chunk_base(k), CHUNK)], sem_w[b])

        guarded(0, lambda: load_and_gather(0))
        for k in range(max_k):
            guarded(k, lambda k=k: finish_chunk(k))
            if k + 1 < max_k:
                guarded(k + 1, lambda k=k: load_and_gather(k + 1))

        def drain(b):
            pltpu.make_async_copy(
                rows_v[b], out_hbm.at[pl.ds(0, CHUNK)], sem_w[b]).wait()

        drain(0)
        drain(1)

    return gather_kernel(input, idx)


# speedup vs baseline: 4.8256x; 3.7025x over previous
import functools

import jax
import jax.numpy as jnp
from jax import lax
from jax.experimental import pallas as pl
from jax.experimental.pallas import tpu as pltpu
from jax.experimental.pallas import tpu_sc as plsc

NUM_CORES = 2
NUM_SUBCORES = 16
NUM_WORKERS = NUM_CORES * NUM_SUBCORES
CHUNK = 400


def kernel(input, node_segment):
    n = node_segment.shape[0]
    d = input.shape[1]
    num_chunks = n // CHUNK
    assert num_chunks * CHUNK == n
    max_k = -(-num_chunks // NUM_WORKERS)
    assert num_chunks // NUM_WORKERS >= 2

    idx = node_segment.astype(jnp.int32)
    mesh = plsc.VectorSubcoreMesh(core_axis_name="c", subcore_axis_name="s")

    @functools.partial(
        pl.kernel,
        out_type=jax.ShapeDtypeStruct((n, d), input.dtype),
        mesh=mesh,
        scratch_types=[
            pltpu.VMEM((CHUNK,), jnp.int32),
            pltpu.VMEM((CHUNK,), jnp.int32),
            pltpu.VMEM((CHUNK, d), jnp.float32),
            pltpu.VMEM((CHUNK, d), jnp.float32),
            pltpu.VMEM_SHARED((input.shape[0], d), jnp.float32),
            pltpu.SemaphoreType.DMA,
            pltpu.SemaphoreType.DMA,
            pltpu.SemaphoreType.DMA,
            pltpu.SemaphoreType.DMA,
        ],
    )
    def gather_kernel(table_hbm, idx_hbm, out_hbm, idx_v0, idx_v1,
                      rows_v0, rows_v1, table_sh, sem_g0, sem_g1, sem_w0,
                      sem_w1):
        sid = lax.axis_index("s")
        wid = sid * NUM_CORES + lax.axis_index("c")

        v = input.shape[0]
        rows_per_sub = v // NUM_SUBCORES
        assert rows_per_sub * NUM_SUBCORES == v
        pltpu.sync_copy(table_hbm.at[pl.ds(sid * rows_per_sub, rows_per_sub)],
                        table_sh.at[pl.ds(sid * rows_per_sub, rows_per_sub)])
        plsc.subcore_barrier()
        idx_v = (idx_v0, idx_v1)
        rows_v = (rows_v0, rows_v1)
        sem_g = (sem_g0, sem_g1)
        sem_w = (sem_w0, sem_w1)

        def chunk_base(k):
            return (wid + k * NUM_WORKERS) * CHUNK

        def have_chunk(k):
            return (k + 1) * NUM_WORKERS <= num_chunks

        def guarded(k, fn):
            if have_chunk(k):
                fn()
            else:
                pl.when(wid + k * NUM_WORKERS < num_chunks)(fn)

        def load_and_gather(k):
            b = k & 1
            pltpu.sync_copy(idx_hbm.at[pl.ds(chunk_base(k), CHUNK)], idx_v[b])
            if k >= 2:
                pltpu.make_async_copy(
                    rows_v[b], out_hbm.at[pl.ds(0, CHUNK)], sem_w[b]).wait()
            pltpu.async_copy(table_sh.at[idx_v[b]], rows_v[b], sem_g[b])

        def finish_chunk(k):
            b = k & 1
            pltpu.make_async_copy(
                table_sh.at[idx_v[b]], rows_v[b], sem_g[b]).wait()
            pltpu.async_copy(
                rows_v[b], out_hbm.at[pl.ds(chunk_base(k), CHUNK)], sem_w[b])

        guarded(0, lambda: load_and_gather(0))
        for k in range(max_k):
            guarded(k, lambda k=k: finish_chunk(k))
            if k + 1 < max_k:
                guarded(k + 1, lambda k=k: load_and_gather(k + 1))

        def drain(b):
            pltpu.make_async_copy(
                rows_v[b], out_hbm.at[pl.ds(0, CHUNK)], sem_w[b]).wait()

        drain(0)
        drain(1)

    return gather_kernel(input, idx)


# speedup vs baseline: 5.6458x; 1.1700x over previous
import functools

import jax
import jax.numpy as jnp
from jax import lax
from jax.experimental import pallas as pl
from jax.experimental.pallas import tpu as pltpu
from jax.experimental.pallas import tpu_sc as plsc

NUM_CORES = 2
NUM_SUBCORES = 16
NUM_WORKERS = NUM_CORES * NUM_SUBCORES
SPAN = 3200
CHUNK = 200
NBUF = 4
DEPTH = 3


def kernel(input, node_segment):
    n = node_segment.shape[0]
    v, d = input.shape
    cps = SPAN // CHUNK
    full_workers = n // SPAN
    rem = n - full_workers * SPAN
    rem_chunks = rem // CHUNK
    assert SPAN % CHUNK == 0 and rem % CHUNK == 0
    assert SPAN % 8 == 0 and CHUNK % 8 == 0
    assert full_workers + (1 if rem else 0) == NUM_WORKERS
    assert min(cps, rem_chunks if rem else cps) >= NBUF

    idx = node_segment.astype(jnp.int32)
    mesh = plsc.VectorSubcoreMesh(core_axis_name="c", subcore_axis_name="s")

    @functools.partial(
        pl.kernel,
        out_type=jax.ShapeDtypeStruct((n, d), input.dtype),
        mesh=mesh,
        scratch_types=[
            pltpu.VMEM((SPAN,), jnp.int32),
            pltpu.VMEM((NBUF, CHUNK, d), jnp.float32),
            pltpu.VMEM_SHARED((v, d), jnp.float32),
            pltpu.SemaphoreType.DMA,
            pltpu.SemaphoreType.DMA((NBUF,)),
            pltpu.SemaphoreType.DMA((NBUF,)),
        ],
    )
    def gather_kernel(table_hbm, idx_hbm, out_hbm, idx_all, rows_v, table_sh,
                      sem_i, sem_g, sem_w):
        sid = lax.axis_index("s")
        wid = sid * NUM_CORES + lax.axis_index("c")
        base = wid * SPAN

        @pl.when(wid < full_workers)
        def _():
            pltpu.async_copy(idx_hbm.at[pl.ds(base, SPAN)],
                             idx_all.at[pl.ds(0, SPAN)], sem_i)

        if rem:
            @pl.when(wid == full_workers)
            def _():
                pltpu.async_copy(idx_hbm.at[pl.ds(base, rem)],
                                 idx_all.at[pl.ds(0, rem)], sem_i)

        rows_per_sub = v // NUM_SUBCORES
        assert rows_per_sub * NUM_SUBCORES == v
        pltpu.sync_copy(table_hbm.at[pl.ds(sid * rows_per_sub, rows_per_sub)],
                        table_sh.at[pl.ds(sid * rows_per_sub, rows_per_sub)])
        plsc.subcore_barrier()

        @pl.when(wid < full_workers)
        def _():
            pltpu.make_async_copy(idx_hbm.at[pl.ds(base, SPAN)],
                                  idx_all.at[pl.ds(0, SPAN)], sem_i).wait()

        if rem:
            @pl.when(wid == full_workers)
            def _():
                pltpu.make_async_copy(idx_hbm.at[pl.ds(base, rem)],
                                      idx_all.at[pl.ds(0, rem)], sem_i).wait()

        def guarded(k, fn):
            if k < (rem_chunks if rem else cps):
                fn()
            else:
                pl.when(wid < full_workers)(fn)

        def start_gather(k):
            b = k % NBUF
            pltpu.async_copy(
                table_sh.at[idx_all.at[pl.ds(k * CHUNK, CHUNK)]],
                rows_v.at[b], sem_g.at[b])

        def wait_write(k):
            b = k % NBUF
            pltpu.make_async_copy(
                rows_v.at[b], out_hbm.at[pl.ds(0, CHUNK)], sem_w.at[b]).wait()

        def finish_chunk(k):
            b = k % NBUF
            pltpu.make_async_copy(
                table_sh.at[idx_all.at[pl.ds(k * CHUNK, CHUNK)]],
                rows_v.at[b], sem_g.at[b]).wait()
            pltpu.async_copy(
                rows_v.at[b], out_hbm.at[pl.ds(base + k * CHUNK, CHUNK)],
                sem_w.at[b])

        for k in range(min(DEPTH, cps)):
            guarded(k, lambda k=k: start_gather(k))
        for k in range(cps):
            guarded(k, lambda k=k: finish_chunk(k))
            j = k + DEPTH
            if j < cps:
                def advance(j=j):
                    if j >= NBUF:
                        wait_write(j - NBUF)
                    start_gather(j)
                guarded(j, advance)

        for b in range(NBUF):
            wait_write(b)

    return gather_kernel(input, idx)


# speedup vs baseline: 5.6488x; 1.0005x over previous
import functools

import jax
import jax.numpy as jnp
from jax import lax
from jax.experimental import pallas as pl
from jax.experimental.pallas import tpu as pltpu
from jax.experimental.pallas import tpu_sc as plsc

NUM_CORES = 2
NUM_SUBCORES = 16
NUM_WORKERS = NUM_CORES * NUM_SUBCORES
SPAN = 3200
CHUNK = 160
NBUF = 5
DEPTH = 4


def kernel(input, node_segment):
    n = node_segment.shape[0]
    v, d = input.shape
    cps = SPAN // CHUNK
    full_workers = n // SPAN
    rem = n - full_workers * SPAN
    rem_chunks = rem // CHUNK
    assert SPAN % CHUNK == 0 and rem % CHUNK == 0
    assert SPAN % 8 == 0 and CHUNK % 8 == 0
    assert full_workers + (1 if rem else 0) == NUM_WORKERS
    assert min(cps, rem_chunks if rem else cps) >= NBUF

    idx = node_segment.astype(jnp.int32)
    mesh = plsc.VectorSubcoreMesh(core_axis_name="c", subcore_axis_name="s")

    @functools.partial(
        pl.kernel,
        out_type=jax.ShapeDtypeStruct((n, d), input.dtype),
        mesh=mesh,
        scratch_types=[
            pltpu.VMEM((SPAN,), jnp.int32),
            pltpu.VMEM((NBUF, CHUNK, d), jnp.float32),
            pltpu.VMEM_SHARED((v, d), jnp.float32),
            pltpu.SemaphoreType.DMA,
            pltpu.SemaphoreType.DMA((NBUF,)),
            pltpu.SemaphoreType.DMA((NBUF,)),
        ],
    )
    def gather_kernel(table_hbm, idx_hbm, out_hbm, idx_all, rows_v, table_sh,
                      sem_i, sem_g, sem_w):
        sid = lax.axis_index("s")
        wid = sid * NUM_CORES + lax.axis_index("c")
        base = wid * SPAN

        @pl.when(wid < full_workers)
        def _():
            pltpu.async_copy(idx_hbm.at[pl.ds(base, SPAN)],
                             idx_all.at[pl.ds(0, SPAN)], sem_i)

        if rem:
            @pl.when(wid == full_workers)
            def _():
                pltpu.async_copy(idx_hbm.at[pl.ds(base, rem)],
                                 idx_all.at[pl.ds(0, rem)], sem_i)

        rows_per_sub = v // NUM_SUBCORES
        assert rows_per_sub * NUM_SUBCORES == v
        pltpu.sync_copy(table_hbm.at[pl.ds(sid * rows_per_sub, rows_per_sub)],
                        table_sh.at[pl.ds(sid * rows_per_sub, rows_per_sub)])
        plsc.subcore_barrier()

        @pl.when(wid < full_workers)
        def _():
            pltpu.make_async_copy(idx_hbm.at[pl.ds(base, SPAN)],
                                  idx_all.at[pl.ds(0, SPAN)], sem_i).wait()

        if rem:
            @pl.when(wid == full_workers)
            def _():
                pltpu.make_async_copy(idx_hbm.at[pl.ds(base, rem)],
                                      idx_all.at[pl.ds(0, rem)], sem_i).wait()

        def guarded(k, fn):
            if k < (rem_chunks if rem else cps):
                fn()
            else:
                pl.when(wid < full_workers)(fn)

        def start_gather(k):
            b = k % NBUF
            pltpu.async_copy(
                table_sh.at[idx_all.at[pl.ds(k * CHUNK, CHUNK)]],
                rows_v.at[b], sem_g.at[b])

        def wait_write(k):
            b = k % NBUF
            pltpu.make_async_copy(
                rows_v.at[b], out_hbm.at[pl.ds(0, CHUNK)], sem_w.at[b]).wait()

        def finish_chunk(k):
            b = k % NBUF
            pltpu.make_async_copy(
                table_sh.at[idx_all.at[pl.ds(k * CHUNK, CHUNK)]],
                rows_v.at[b], sem_g.at[b]).wait()
            pltpu.async_copy(
                rows_v.at[b], out_hbm.at[pl.ds(base + k * CHUNK, CHUNK)],
                sem_w.at[b])

        for k in range(min(DEPTH, cps)):
            guarded(k, lambda k=k: start_gather(k))
        for k in range(cps):
            guarded(k, lambda k=k: finish_chunk(k))
            j = k + DEPTH
            if j < cps:
                def advance(j=j):
                    if j >= NBUF:
                        wait_write(j - NBUF)
                    start_gather(j)
                guarded(j, advance)

        for b in range(NBUF):
            wait_write(b)

    return gather_kernel(input, idx)
